# baseline (speedup 1.0000x reference)
import functools

import jax
import jax.numpy as jnp
from jax import lax
from jax.experimental import pallas as pl
from jax.experimental.pallas import tpu as pltpu
from jax.experimental.pallas import tpu_sc as plsc

F = 2048
E = 16
T = 8192
L = 16
NC = 2
NS = 16
NW = NC * NS

T_SC = 512
SPLIT = T - T_SC
TPW = T_SC // NW
BLOCK_T = SPLIT // 8


def _tc_body(x_ref, w_ref, b_ref, o_ref):
    logits = lax.dot_general(
        x_ref[...], w_ref[...], (((1,), (1,)), ((), ())),
        preferred_element_type=jnp.float32,
    ) + b_ref[...]
    m = jnp.max(logits, axis=-1, keepdims=True)
    e = jnp.exp(logits - m)
    o_ref[...] = e / jnp.sum(e, axis=-1, keepdims=True)


def _tc_router(x, W, b2):
    return pl.pallas_call(
        _tc_body,
        grid=(SPLIT // BLOCK_T,),
        in_specs=[
            pl.BlockSpec((BLOCK_T, F), lambda i: (i, 0)),
            pl.BlockSpec((E, F), lambda i: (0, 0)),
            pl.BlockSpec((1, E), lambda i: (0, 0)),
        ],
        out_specs=pl.BlockSpec((BLOCK_T, E), lambda i: (i, 0)),
        out_shape=jax.ShapeDtypeStruct((SPLIT, E), jnp.float32),
    )(x, W, b2)


def _lane_perm(v, idx):
    return lax.gather(
        v, idx[:, None],
        dimension_numbers=lax.GatherDimensionNumbers(
            offset_dims=(), collapsed_slice_dims=(0,), start_index_map=(0,)),
        slice_sizes=(1,),
        mode=lax.GatherScatterMode.PROMISE_IN_BOUNDS,
    )


def _sc_body(x_hbm, wt_hbm, b_hbm, out_hbm, wt_v, b_v, xb, obuf, sem_x, sem_w):
    wid = lax.axis_index("s") * NC + lax.axis_index("c")
    base = wid * TPW
    xcp = pltpu.make_async_copy(x_hbm.at[pl.ds(base, TPW)], xb, sem_x)
    xcp.start()
    wcp = pltpu.make_async_copy(wt_hbm, wt_v, sem_w)
    wcp.start()
    pltpu.sync_copy(b_hbm, b_v)
    bvec = b_v[...]
    wcp.wait()
    xcp.wait()

    zero = jnp.zeros((L,), jnp.float32)
    lanes = lax.iota(jnp.int32, L)
    for t in range(TPW):
        def k_body(k, accs):
            acc_a, acc_b = accs
            xv = xb[t, pl.ds(k * L, L)]
            for j in range(L):
                w = wt_v[k * L + j, :]
                if j % 2 == 0:
                    acc_a = acc_a + xv[j] * w
                else:
                    acc_b = acc_b + xv[j] * w
            return (acc_a, acc_b)

        acc_a, acc_b = lax.fori_loop(0, F // L, k_body, (bvec, zero))
        acc = acc_a + acc_b
        m = acc
        for st in (1, 2, 4, 8):
            m = jnp.maximum(m, _lane_perm(m, lanes ^ st))
        e = jnp.exp(acc - m)
        s = e
        for st in (1, 2, 4, 8):
            s = s + _lane_perm(s, lanes ^ st)
        obuf[t, :] = e / s

    pltpu.sync_copy(obuf, out_hbm.at[pl.ds(base, TPW)])


def _sc_router(x_sc, wt, b):
    mesh = plsc.VectorSubcoreMesh(core_axis_name="c", subcore_axis_name="s")
    return functools.partial(
        pl.kernel,
        out_type=jax.ShapeDtypeStruct((T_SC, E), jnp.float32),
        mesh=mesh,
        scratch_types=[
            pltpu.VMEM((F, E), jnp.float32),
            pltpu.VMEM((L,), jnp.float32),
            pltpu.VMEM((TPW, F), jnp.float32),
            pltpu.VMEM((TPW, E), jnp.float32),
            pltpu.SemaphoreType.DMA,
            pltpu.SemaphoreType.DMA,
        ],
        compiler_params=pltpu.CompilerParams(
            use_tc_tiling_on_sc=False, needs_layout_passes=False),
    )(_sc_body)(x_sc, wt, b)


def kernel(x, W, b):
    x_sc = lax.slice(x, (SPLIT, 0), (T, F))
    sc_out = _sc_router(x_sc, W.T, b)
    tc_out = _tc_router(x, W, b.reshape(1, E))
    return jnp.concatenate([tc_out, sc_out], axis=0)


# speedup vs baseline: 1.0228x; 1.0228x over previous
import functools

import jax
import jax.numpy as jnp
from jax import lax
from jax.experimental import pallas as pl
from jax.experimental.pallas import tpu as pltpu
from jax.experimental.pallas import tpu_sc as plsc

F = 2048
E = 16
T = 8192
L = 16
NC = 2
NS = 16
NW = NC * NS

T_SC = 512
SPLIT = T - T_SC
TPW = T_SC // NW
BLOCK_T = SPLIT // 8


def _tc_body(x_ref, w_ref, b_ref, o_ref):
    logits = lax.dot_general(
        x_ref[...], w_ref[...], (((1,), (1,)), ((), ())),
        preferred_element_type=jnp.float32,
    ) + b_ref[...]
    m = jnp.max(logits, axis=-1, keepdims=True)
    e = jnp.exp(logits - m)
    o_ref[...] = e / jnp.sum(e, axis=-1, keepdims=True)


def _tc_router(x, W, b2):
    return pl.pallas_call(
        _tc_body,
        grid=(SPLIT // BLOCK_T,),
        in_specs=[
            pl.BlockSpec((BLOCK_T, F), lambda i: (i, 0)),
            pl.BlockSpec((E, F), lambda i: (0, 0)),
            pl.BlockSpec((1, E), lambda i: (0, 0)),
        ],
        out_specs=pl.BlockSpec((BLOCK_T, E), lambda i: (i, 0)),
        out_shape=jax.ShapeDtypeStruct((SPLIT, E), jnp.float32),
    )(x, W, b2)


def _lane_perm(v, idx):
    return lax.gather(
        v, idx[:, None],
        dimension_numbers=lax.GatherDimensionNumbers(
            offset_dims=(), collapsed_slice_dims=(0,), start_index_map=(0,)),
        slice_sizes=(1,),
        mode=lax.GatherScatterMode.PROMISE_IN_BOUNDS,
    )


def _sc_body(x_hbm, wt_hbm, b_hbm, out_hbm, wt_v, b_v, xb, obuf, sem_x, sem_w):
    wid = lax.axis_index("s") * NC + lax.axis_index("c")
    base = wid * TPW
    xcp = pltpu.make_async_copy(x_hbm.at[pl.ds(base, TPW)], xb, sem_x)
    xcp.start()
    wcp = pltpu.make_async_copy(wt_hbm, wt_v, sem_w)
    wcp.start()
    pltpu.sync_copy(b_hbm, b_v)
    bvec = b_v[...]
    wcp.wait()
    xcp.wait()

    zero = jnp.zeros((L,), jnp.float32)
    lanes = lax.iota(jnp.int32, L)

    def tok_body(t, _):
        def k_body(k, accs):
            acc_a, acc_b = accs
            xv = xb[t, pl.ds(k * L, L)]
            for j in range(L):
                w = wt_v[k * L + j, :]
                if j % 2 == 0:
                    acc_a = acc_a + xv[j] * w
                else:
                    acc_b = acc_b + xv[j] * w
            return (acc_a, acc_b)

        acc_a, acc_b = lax.fori_loop(0, F // L, k_body, (bvec, zero))
        acc = acc_a + acc_b
        m = acc
        for st in (1, 2, 4, 8):
            m = jnp.maximum(m, _lane_perm(m, lanes ^ st))
        e = jnp.exp(acc - m)
        s = e
        for st in (1, 2, 4, 8):
            s = s + _lane_perm(s, lanes ^ st)
        obuf[t, :] = e / s
        return 0

    lax.fori_loop(0, TPW, tok_body, 0)
    pltpu.sync_copy(obuf, out_hbm.at[pl.ds(base, TPW)])


def _sc_router(x_sc, wt, b):
    mesh = plsc.VectorSubcoreMesh(core_axis_name="c", subcore_axis_name="s")
    return functools.partial(
        pl.kernel,
        out_type=jax.ShapeDtypeStruct((T_SC, E), jnp.float32),
        mesh=mesh,
        scratch_types=[
            pltpu.VMEM((F, E), jnp.float32),
            pltpu.VMEM((L,), jnp.float32),
            pltpu.VMEM((TPW, F), jnp.float32),
            pltpu.VMEM((TPW, E), jnp.float32),
            pltpu.SemaphoreType.DMA,
            pltpu.SemaphoreType.DMA,
        ],
        compiler_params=pltpu.CompilerParams(
            use_tc_tiling_on_sc=False, needs_layout_passes=False),
    )(_sc_body)(x_sc, wt, b)


def kernel(x, W, b):
    x_sc = lax.slice(x, (SPLIT, 0), (T, F))
    sc_out = _sc_router(x_sc, W.T, b)
    tc_out = _tc_router(x, W, b.reshape(1, E))
    return jnp.concatenate([tc_out, sc_out], axis=0)


# speedup vs baseline: 1.0999x; 1.0753x over previous
import functools

import jax
import jax.numpy as jnp
from jax import lax
from jax.experimental import pallas as pl
from jax.experimental.pallas import tpu as pltpu
from jax.experimental.pallas import tpu_sc as plsc

F = 2048
E = 16
T = 8192
L = 16
NC = 2
NS = 16
NW = NC * NS

T_SC = 256
SPLIT = T - T_SC
TPW = T_SC // NW
BLOCK_T = SPLIT // 8


def _tc_body(x_ref, w_ref, b_ref, o_ref):
    logits = lax.dot_general(
        x_ref[...], w_ref[...], (((1,), (1,)), ((), ())),
        preferred_element_type=jnp.float32,
    ) + b_ref[...]
    m = jnp.max(logits, axis=-1, keepdims=True)
    e = jnp.exp(logits - m)
    o_ref[...] = e / jnp.sum(e, axis=-1, keepdims=True)


def _tc_router(x, W, b2):
    return pl.pallas_call(
        _tc_body,
        grid=(SPLIT // BLOCK_T,),
        in_specs=[
            pl.BlockSpec((BLOCK_T, F), lambda i: (i, 0)),
            pl.BlockSpec((E, F), lambda i: (0, 0)),
            pl.BlockSpec((1, E), lambda i: (0, 0)),
        ],
        out_specs=pl.BlockSpec((BLOCK_T, E), lambda i: (i, 0)),
        out_shape=jax.ShapeDtypeStruct((SPLIT, E), jnp.float32),
    )(x, W, b2)


def _lane_perm(v, idx):
    return lax.gather(
        v, idx[:, None],
        dimension_numbers=lax.GatherDimensionNumbers(
            offset_dims=(), collapsed_slice_dims=(0,), start_index_map=(0,)),
        slice_sizes=(1,),
        mode=lax.GatherScatterMode.PROMISE_IN_BOUNDS,
    )


def _sc_body(x_hbm, wt_hbm, b_hbm, out_hbm, wt_v, b_v, xb, obuf, sem_x, sem_w):
    wid = lax.axis_index("s") * NC + lax.axis_index("c")
    base = wid * TPW
    xcp = pltpu.make_async_copy(x_hbm.at[pl.ds(base, TPW)], xb, sem_x)
    xcp.start()
    wcp = pltpu.make_async_copy(wt_hbm, wt_v, sem_w)
    wcp.start()
    pltpu.sync_copy(b_hbm, b_v)
    bvec = b_v[...]
    wcp.wait()
    xcp.wait()

    zero = jnp.zeros((L,), jnp.float32)
    lanes = lax.iota(jnp.int32, L)

    for t in range(TPW):
        def k_body(k, accs):
            acc_a, acc_b = accs
            xv = xb[t, pl.ds(k * L, L)]
            for j in range(L):
                w = wt_v[k * L + j, :]
                if j % 2 == 0:
                    acc_a = acc_a + xv[j] * w
                else:
                    acc_b = acc_b + xv[j] * w
            return (acc_a, acc_b)

        acc_a, acc_b = lax.fori_loop(0, F // L, k_body, (bvec, zero))
        acc = acc_a + acc_b
        m = acc
        for st in (1, 2, 4, 8):
            m = jnp.maximum(m, _lane_perm(m, lanes ^ st))
        e = jnp.exp(acc - m)
        s = e
        for st in (1, 2, 4, 8):
            s = s + _lane_perm(s, lanes ^ st)
        obuf[t, :] = e / s

    pltpu.sync_copy(obuf, out_hbm.at[pl.ds(base, TPW)])


def _sc_router(x_sc, wt, b):
    mesh = plsc.VectorSubcoreMesh(core_axis_name="c", subcore_axis_name="s")
    return functools.partial(
        pl.kernel,
        out_type=jax.ShapeDtypeStruct((T_SC, E), jnp.float32),
        mesh=mesh,
        scratch_types=[
            pltpu.VMEM((F, E), jnp.float32),
            pltpu.VMEM((L,), jnp.float32),
            pltpu.VMEM((TPW, F), jnp.float32),
            pltpu.VMEM((TPW, E), jnp.float32),
            pltpu.SemaphoreType.DMA,
            pltpu.SemaphoreType.DMA,
        ],
        compiler_params=pltpu.CompilerParams(
            use_tc_tiling_on_sc=False, needs_layout_passes=False),
    )(_sc_body)(x_sc, wt, b)


def kernel(x, W, b):
    tc_out = _tc_router(x, W, b.reshape(1, E))
    x_sc = lax.slice(x, (SPLIT, 0), (T, F))
    sc_out = _sc_router(x_sc, W.T, b)
    return jnp.concatenate([tc_out, sc_out], axis=0)


# speedup vs baseline: 1.1065x; 1.0060x over previous
import functools

import jax
import jax.numpy as jnp
from jax import lax
from jax.experimental import pallas as pl
from jax.experimental.pallas import tpu as pltpu
from jax.experimental.pallas import tpu_sc as plsc

F = 2048
E = 16
T = 8192
L = 16
NC = 2
NS = 16
NW = NC * NS

T_SC = 384
SPLIT = T - T_SC
TPW = T_SC // NW
BLOCK_T = SPLIT // 8


def _tc_body(x_ref, w_ref, b_ref, o_ref):
    logits = lax.dot_general(
        x_ref[...], w_ref[...], (((1,), (1,)), ((), ())),
        preferred_element_type=jnp.float32,
    ) + b_ref[...]
    m = jnp.max(logits, axis=-1, keepdims=True)
    e = jnp.exp(logits - m)
    o_ref[...] = e / jnp.sum(e, axis=-1, keepdims=True)


def _tc_router(x, W, b2):
    return pl.pallas_call(
        _tc_body,
        grid=(SPLIT // BLOCK_T,),
        in_specs=[
            pl.BlockSpec((BLOCK_T, F), lambda i: (i, 0)),
            pl.BlockSpec((E, F), lambda i: (0, 0)),
            pl.BlockSpec((1, E), lambda i: (0, 0)),
        ],
        out_specs=pl.BlockSpec((BLOCK_T, E), lambda i: (i, 0)),
        out_shape=jax.ShapeDtypeStruct((SPLIT, E), jnp.float32),
    )(x, W, b2)


def _lane_perm(v, idx):
    return lax.gather(
        v, idx[:, None],
        dimension_numbers=lax.GatherDimensionNumbers(
            offset_dims=(), collapsed_slice_dims=(0,), start_index_map=(0,)),
        slice_sizes=(1,),
        mode=lax.GatherScatterMode.PROMISE_IN_BOUNDS,
    )


def _sc_body(x_hbm, wt_hbm, b_hbm, out_hbm, wt_v, b_v, xb, obuf, sem_x, sem_w):
    wid = lax.axis_index("s") * NC + lax.axis_index("c")
    base = wid * TPW
    xcp = pltpu.make_async_copy(x_hbm.at[pl.ds(base, TPW)], xb, sem_x)
    xcp.start()
    wcp = pltpu.make_async_copy(wt_hbm, wt_v, sem_w)
    wcp.start()
    pltpu.sync_copy(b_hbm, b_v)
    bvec = b_v[...]
    wcp.wait()
    xcp.wait()

    zero = jnp.zeros((L,), jnp.float32)
    lanes = lax.iota(jnp.int32, L)

    for t in range(TPW):
        def k_body(k, accs):
            acc_a, acc_b = accs
            xv = xb[t, pl.ds(k * L, L)]
            for j in range(L):
                w = wt_v[k * L + j, :]
                if j % 2 == 0:
                    acc_a = acc_a + xv[j] * w
                else:
                    acc_b = acc_b + xv[j] * w
            return (acc_a, acc_b)

        acc_a, acc_b = lax.fori_loop(0, F // L, k_body, (bvec, zero))
        acc = acc_a + acc_b
        m = acc
        for st in (1, 2, 4, 8):
            m = jnp.maximum(m, _lane_perm(m, lanes ^ st))
        e = jnp.exp(acc - m)
        s = e
        for st in (1, 2, 4, 8):
            s = s + _lane_perm(s, lanes ^ st)
        obuf[t, :] = e / s

    pltpu.sync_copy(obuf, out_hbm.at[pl.ds(base, TPW)])


def _sc_router(x_sc, wt, b):
    mesh = plsc.VectorSubcoreMesh(core_axis_name="c", subcore_axis_name="s")
    return functools.partial(
        pl.kernel,
        out_type=jax.ShapeDtypeStruct((T_SC, E), jnp.float32),
        mesh=mesh,
        scratch_types=[
            pltpu.VMEM((F, E), jnp.float32),
            pltpu.VMEM((L,), jnp.float32),
            pltpu.VMEM((TPW, F), jnp.float32),
            pltpu.VMEM((TPW, E), jnp.float32),
            pltpu.SemaphoreType.DMA,
            pltpu.SemaphoreType.DMA,
        ],
        compiler_params=pltpu.CompilerParams(
            use_tc_tiling_on_sc=False, needs_layout_passes=False),
    )(_sc_body)(x_sc, wt, b)


def kernel(x, W, b):
    tc_out = _tc_router(x, W, b.reshape(1, E))
    x_sc = lax.slice(x, (SPLIT, 0), (T, F))
    sc_out = _sc_router(x_sc, W.T, b)
    return jnp.concatenate([tc_out, sc_out], axis=0)
